# Initial kernel scaffold; baseline (speedup 1.0000x reference)
import functools

import jax
import jax.numpy as jnp
from jax import lax
from jax.experimental import pallas as pl
from jax.experimental.pallas import tpu as pltpu
from jax.experimental.pallas import tpu_sc as plsc

N = 10000
E = 320000
D = 128
HID = 128
G = 128

NC, NS = 2, 16
EPC = E // NC
EPT = EPC // NS
CH = 80
NCHUNK = EPT // CH
RPT = N // NS
ZR = 125

PG = 40
PR = PG * 8


def _sc_edge_aggregate(x, src, dst):
  mesh = plsc.VectorSubcoreMesh(core_axis_name="c", subcore_axis_name="s")

  @functools.partial(
      pl.kernel,
      out_type=(jax.ShapeDtypeStruct((NC, N, D), jnp.float32),
                jax.ShapeDtypeStruct((NC, N, 16), jnp.float32)),
      mesh=mesh,
      scratch_types=[
          pltpu.VMEM((CH,), jnp.int32),
          pltpu.VMEM((CH,), jnp.int32),
          pltpu.VMEM((CH, D), jnp.float32),
          pltpu.VMEM((CH, 16), jnp.float32),
          pltpu.VMEM((ZR, D), jnp.float32),
          pltpu.VMEM((RPT, 16), jnp.float32),
          pltpu.VMEM_SHARED((N, D), jnp.float32),
          pltpu.VMEM_SHARED((N, 16), jnp.float32),
          pltpu.SemaphoreType.DMA,
      ],
  )
  def k(x_hbm, src_hbm, dst_hbm, agg_hbm, deg_hbm,
        isrc, idst, rows, ones, zbuf, zdeg, sh_agg, sh_deg, sem):
    c = lax.axis_index("c")
    s = lax.axis_index("s")

    @pl.loop(0, CH)
    def _(i):
      ones[i, pl.ds(0, 16)] = jnp.ones((16,), jnp.float32)

    @pl.loop(0, ZR)
    def _(i):
      @pl.loop(0, D // 16)
      def _(j):
        zbuf[i, pl.ds(j * 16, 16)] = jnp.zeros((16,), jnp.float32)

    @pl.loop(0, RPT)
    def _(i):
      zdeg[i, pl.ds(0, 16)] = jnp.zeros((16,), jnp.float32)

    r0 = s * RPT

    @pl.loop(0, RPT // ZR)
    def _(j):
      pltpu.sync_copy(zbuf, sh_agg.at[pl.ds(r0 + j * ZR, ZR)])

    pltpu.sync_copy(zdeg, sh_deg.at[pl.ds(r0, RPT)])
    plsc.subcore_barrier()

    base_t = c * EPC + s * EPT

    @pl.loop(0, NCHUNK)
    def _(kk):
      b = base_t + kk * CH
      pltpu.sync_copy(src_hbm.at[pl.ds(b, CH)], isrc)
      pltpu.sync_copy(dst_hbm.at[pl.ds(b, CH)], idst)
      pltpu.async_copy(x_hbm.at[isrc], rows, sem).wait()
      pltpu.sync_copy(rows, sh_agg.at[idst], add=True)
      pltpu.sync_copy(ones, sh_deg.at[idst], add=True)

    plsc.subcore_barrier()
    pltpu.sync_copy(sh_agg.at[pl.ds(r0, RPT)], agg_hbm.at[c, pl.ds(r0, RPT)])
    pltpu.sync_copy(sh_deg.at[pl.ds(r0, RPT)], deg_hbm.at[c, pl.ds(r0, RPT)])

  return k(x, src, dst)


def _tc_h(x, agg, deg, wl, bl, wr):
  BN = 1000

  def body(x_ref, a_ref, d_ref, wl_ref, bl_ref, wr_ref, h_ref):
    a = a_ref[0] + a_ref[1]
    d = d_ref[0][:, :1] + d_ref[1][:, :1]
    mean = a / jnp.clip(d, 1.0, None)
    acc = lax.dot_general(mean, wl_ref[...], (((1,), (1,)), ((), ())),
                          preferred_element_type=jnp.float32)
    acc += bl_ref[...]
    acc += lax.dot_general(x_ref[...], wr_ref[...], (((1,), (1,)), ((), ())),
                           preferred_element_type=jnp.float32)
    h_ref[...] = jnp.maximum(acc, 0.0)

  return pl.pallas_call(
      body,
      grid=(N // BN,),
      in_specs=[
          pl.BlockSpec((BN, D), lambda i: (i, 0)),
          pl.BlockSpec((NC, BN, D), lambda i: (0, i, 0)),
          pl.BlockSpec((NC, BN, 16), lambda i: (0, i, 0)),
          pl.BlockSpec((HID, D), lambda i: (0, 0)),
          pl.BlockSpec((1, HID), lambda i: (0, 0)),
          pl.BlockSpec((HID, D), lambda i: (0, 0)),
      ],
      out_specs=pl.BlockSpec((BN, HID), lambda i: (i, 0)),
      out_shape=jax.ShapeDtypeStruct((N, HID), jnp.float32),
  )(x, agg, deg, wl, bl, wr)


def _sc_pool(h, batch2d):
  mesh = plsc.VectorSubcoreMesh(core_axis_name="c", subcore_axis_name="s")

  @functools.partial(
      pl.kernel,
      out_type=jax.ShapeDtypeStruct((NC * NS, G, HID), jnp.float32),
      mesh=mesh,
      scratch_types=[
          pltpu.VMEM((PR, HID), jnp.float32),
          pltpu.VMEM((G, HID), jnp.float32),
          pltpu.SMEM((PG, 8), jnp.int32),
      ],
  )
  def k(h_hbm, b_hbm, out_hbm, hbuf, pool, bsm):
    c = lax.axis_index("c")
    s = lax.axis_index("s")
    wid = s * NC + c

    @pl.loop(0, G)
    def _(i):
      @pl.loop(0, HID // 16)
      def _(j):
        pool[i, pl.ds(j * 16, 16)] = jnp.full((16,), -jnp.inf, jnp.float32)

    g0 = jnp.minimum(wid * PG, N // 8 - PG)
    pltpu.sync_copy(h_hbm.at[pl.ds(g0 * 8, PR)], hbuf)
    pltpu.sync_copy(b_hbm.at[pl.ds(g0, PG)], bsm)

    @pl.loop(0, PG)
    def _(gi):
      @pl.loop(0, 8)
      def _(ri):
        g = bsm[gi, ri]
        r = gi * 8 + ri

        @pl.loop(0, HID // 16)
        def _(j):
          v = hbuf[r, pl.ds(j * 16, 16)]
          cur = pool[g, pl.ds(j * 16, 16)]
          pool[g, pl.ds(j * 16, 16)] = jnp.maximum(cur, v)

    pltpu.sync_copy(pool, out_hbm.at[wid])

  return k(h, batch2d)


def _tc_head(pool32, batch2d, x, w0, b0, w1, b1, w2, b2):
  def body(p_ref, b_ref, x_ref, w0_ref, b0_ref, w1_ref, b1_ref, w2_ref,
           b2_ref, o_ref):
    pooled = jnp.max(p_ref[...], axis=0)
    b = b_ref[...].reshape(1, N)
    garange = lax.broadcasted_iota(jnp.int32, (G, 1), 0)
    cnt = jnp.sum((b < garange).astype(jnp.int32), axis=1, keepdims=True)
    root = jnp.clip(cnt, 0, N - 1)
    narange = lax.broadcasted_iota(jnp.int32, (G, N), 1)
    sel = (narange == root).astype(jnp.float32)
    news = lax.dot_general(sel, x_ref[...], (((1,), (0,)), ((), ())),
                           preferred_element_type=jnp.float32)
    news = jnp.maximum(
        lax.dot_general(news, w0_ref[...], (((1,), (1,)), ((), ())),
                        preferred_element_type=jnp.float32) + b0_ref[...],
        0.0)
    cat = jnp.concatenate([news, pooled], axis=-1)
    h1 = jnp.maximum(
        lax.dot_general(cat, w1_ref[...], (((1,), (1,)), ((), ())),
                        preferred_element_type=jnp.float32) + b1_ref[...],
        0.0)
    out = lax.dot_general(h1, w2_ref[...], (((1,), (1,)), ((), ())),
                          preferred_element_type=jnp.float32) + b2_ref[...]
    m = jnp.max(out, axis=-1, keepdims=True)
    z = out - m
    o_ref[...] = z - jnp.log(jnp.sum(jnp.exp(z), axis=-1, keepdims=True))

  return pl.pallas_call(
      body,
      out_shape=jax.ShapeDtypeStruct((G, 2), jnp.float32),
  )(pool32, batch2d, x, w0, b0, w1, b1, w2, b2)


def kernel(x, edge_index, batch, lin_l_w, lin_l_b, lin_r_w,
           lin0_w, lin0_b, lin1_w, lin1_b, lin2_w, lin2_b):
  src = edge_index[0]
  dst = edge_index[1]
  agg, deg = _sc_edge_aggregate(x, src, dst)
  h = _tc_h(x, agg, deg, lin_l_w, lin_l_b.reshape(1, HID), lin_r_w)
  batch2d = batch.reshape(N // 8, 8)
  pool32 = _sc_pool(h, batch2d)
  return _tc_head(pool32, batch2d, x,
                  lin0_w, lin0_b.reshape(1, HID),
                  lin1_w, lin1_b.reshape(1, HID),
                  lin2_w, lin2_b.reshape(1, 2))


# speedup vs baseline: 6.0500x; 6.0500x over previous
import dataclasses
import functools

import jax
import jax.numpy as jnp
from jax import lax
from jax.experimental import pallas as pl
from jax.experimental.pallas import tpu as pltpu
from jax.experimental.pallas import tpu_sc as plsc

N = 10000
E = 320000
D = 128
HID = 128
G = 128

NC, NS = 2, 16
NW = NC * NS
EPC = E // NC
EPT = EPC // NS
CH = 80
NCHUNK = EPT // CH
RSTEP = 624
RSZ = 640

PG = 40
PR = PG * 8


def _sc_edge_aggregate(x, src, dst, zagg):
  mesh = plsc.VectorSubcoreMesh(core_axis_name="c", subcore_axis_name="s")
  cp = pltpu.CompilerParams()
  if "needs_layout_passes" in pltpu.CompilerParams.__dataclass_fields__:
    cp = dataclasses.replace(cp, needs_layout_passes=False)

  @functools.partial(
      pl.kernel,
      compiler_params=cp,
      out_type=(jax.ShapeDtypeStruct((NC, N, D), jnp.float32),
                jax.ShapeDtypeStruct((NW, N), jnp.float32)),
      mesh=mesh,
      scratch_types=[
          pltpu.VMEM((CH,), jnp.int32),
          pltpu.VMEM((CH,), jnp.int32),
          pltpu.VMEM((CH, D), jnp.float32),
          pltpu.VMEM((N,), jnp.float32),
          pltpu.VMEM_SHARED((N, D), jnp.float32),
          pltpu.SemaphoreType.DMA,
      ],
  )
  def k(x_hbm, src_hbm, dst_hbm, zagg_hbm, agg_hbm, deg_hbm,
        isrc, idst, rows, degl, sh_agg, sem):
    c = lax.axis_index("c")
    s = lax.axis_index("s")
    r0 = s * RSTEP

    @pl.loop(0, N // 16)
    def _(i):
      degl[pl.ds(i * 16, 16)] = jnp.zeros((16,), jnp.float32)

    pltpu.sync_copy(zagg_hbm, rows)

    @pl.loop(0, RSZ // CH)
    def _(t):
      pltpu.sync_copy(rows, sh_agg.at[pl.ds(r0 + t * CH, CH)])

    plsc.subcore_barrier()

    base_t = c * EPC + s * EPT
    ones16 = jnp.ones((16,), jnp.float32)

    @pl.loop(0, NCHUNK)
    def _(kk):
      b = base_t + kk * CH
      pltpu.sync_copy(dst_hbm.at[pl.ds(b, CH)], idst)
      pltpu.sync_copy(src_hbm.at[pl.ds(b, CH)], isrc)
      for i in range(CH // 16):
        plsc.addupdate_scatter(degl, [idst[pl.ds(i * 16, 16)]], ones16)
      pltpu.async_copy(x_hbm.at[isrc], rows, sem).wait()
      pltpu.sync_copy(rows, sh_agg.at[idst], add=True)

    pltpu.sync_copy(degl, deg_hbm.at[s * NC + c])
    plsc.subcore_barrier()

    @pl.loop(0, RSZ // CH)
    def _(t):
      rr = r0 + t * CH
      pltpu.sync_copy(sh_agg.at[pl.ds(rr, CH)], rows)
      pltpu.sync_copy(rows, agg_hbm.at[c, pl.ds(rr, CH)])

  return k(x, src, dst, zagg)


def _tc_h(x, agg, deg, wl, bl, wr):
  BN = 1000

  def body(x_ref, a_ref, d_ref, wl_ref, bl_ref, wr_ref, h_ref):
    a = a_ref[0] + a_ref[1]
    d = jnp.sum(d_ref[...], axis=1)[:, None]
    mean = a / jnp.clip(d, 1.0, None)
    acc = lax.dot_general(mean, wl_ref[...], (((1,), (1,)), ((), ())),
                          preferred_element_type=jnp.float32)
    acc += bl_ref[...]
    acc += lax.dot_general(x_ref[...], wr_ref[...], (((1,), (1,)), ((), ())),
                           preferred_element_type=jnp.float32)
    h_ref[...] = jnp.maximum(acc, 0.0)

  return pl.pallas_call(
      body,
      grid=(N // BN,),
      in_specs=[
          pl.BlockSpec((BN, D), lambda i: (i, 0)),
          pl.BlockSpec((NC, BN, D), lambda i: (0, i, 0)),
          pl.BlockSpec((BN, NW), lambda i: (i, 0)),
          pl.BlockSpec((HID, D), lambda i: (0, 0)),
          pl.BlockSpec((1, HID), lambda i: (0, 0)),
          pl.BlockSpec((HID, D), lambda i: (0, 0)),
      ],
      out_specs=pl.BlockSpec((BN, HID), lambda i: (i, 0)),
      out_shape=jax.ShapeDtypeStruct((N, HID), jnp.float32),
  )(x, agg, deg, wl, bl, wr)


def _sc_pool(h, batch1d):
  mesh = plsc.VectorSubcoreMesh(core_axis_name="c", subcore_axis_name="s")

  @functools.partial(
      pl.kernel,
      out_type=jax.ShapeDtypeStruct((NW, G, HID), jnp.float32),
      mesh=mesh,
      scratch_types=[
          pltpu.VMEM((PR, HID), jnp.float32),
          pltpu.VMEM((G, HID), jnp.float32),
          pltpu.VMEM((PR,), jnp.int32),
      ],
  )
  def k(h_hbm, b_hbm, out_hbm, hbuf, pool, bsm):
    c = lax.axis_index("c")
    s = lax.axis_index("s")
    wid = s * NC + c

    @pl.loop(0, G)
    def _(i):
      @pl.loop(0, HID // 16)
      def _(j):
        pool[i, pl.ds(j * 16, 16)] = jnp.full((16,), -jnp.inf, jnp.float32)

    def scan_rows(nblk):
      @pl.loop(0, nblk)
      def _(bi):
        bv = bsm[pl.ds(bi * 16, 16)]
        for ri in range(16):
          g = bv[ri]
          r = bi * 16 + ri

          @pl.loop(0, HID // 16)
          def _(j):
            v = hbuf[r, pl.ds(j * 16, 16)]
            cur = pool[g, pl.ds(j * 16, 16)]
            pool[g, pl.ds(j * 16, 16)] = jnp.maximum(cur, v)

    g0 = 8 * jnp.minimum(wid * (PG // 8), (1248 - PG) // 8)
    pltpu.sync_copy(h_hbm.at[pl.ds(g0 * 8, PR)], hbuf)
    pltpu.sync_copy(b_hbm.at[pl.ds(g0 * 8, PR)], bsm)
    scan_rows(PR // 16)

    @pl.when(wid == NW - 1)
    def _():
      pltpu.sync_copy(h_hbm.at[pl.ds(1248 * 8, 16)], hbuf.at[pl.ds(0, 16)])
      pltpu.sync_copy(b_hbm.at[pl.ds(1248 * 8, 16)], bsm.at[pl.ds(0, 16)])
      scan_rows(1)

    pltpu.sync_copy(pool, out_hbm.at[wid])

  return k(h, batch1d)


def _tc_head(pool32, batch_row, x, w0, b0, w1, b1, w2, b2):
  def body(p_ref, b_ref, x_ref, w0_ref, b0_ref, w1_ref, b1_ref, w2_ref,
           b2_ref, o_ref):
    pooled = jnp.max(p_ref[...], axis=0)
    b = b_ref[...]
    garange = lax.broadcasted_iota(jnp.int32, (G, 1), 0)
    cnt = jnp.sum((b < garange).astype(jnp.int32), axis=1, keepdims=True)
    root = jnp.clip(cnt, 0, N - 1)
    narange = lax.broadcasted_iota(jnp.int32, (G, N), 1)
    sel = (narange == root).astype(jnp.float32)
    news = lax.dot_general(sel, x_ref[...], (((1,), (0,)), ((), ())),
                           preferred_element_type=jnp.float32)
    news = jnp.maximum(
        lax.dot_general(news, w0_ref[...], (((1,), (1,)), ((), ())),
                        preferred_element_type=jnp.float32) + b0_ref[...],
        0.0)
    cat = jnp.concatenate([news, pooled], axis=-1)
    h1 = jnp.maximum(
        lax.dot_general(cat, w1_ref[...], (((1,), (1,)), ((), ())),
                        preferred_element_type=jnp.float32) + b1_ref[...],
        0.0)
    out = lax.dot_general(h1, w2_ref[...], (((1,), (1,)), ((), ())),
                          preferred_element_type=jnp.float32) + b2_ref[...]
    m = jnp.max(out, axis=-1, keepdims=True)
    z = out - m
    o_ref[...] = z - jnp.log(jnp.sum(jnp.exp(z), axis=-1, keepdims=True))

  return pl.pallas_call(
      body,
      out_shape=jax.ShapeDtypeStruct((G, 2), jnp.float32),
  )(pool32, batch_row, x, w0, b0, w1, b1, w2, b2)


def kernel(x, edge_index, batch, lin_l_w, lin_l_b, lin_r_w,
           lin0_w, lin0_b, lin1_w, lin1_b, lin2_w, lin2_b):
  src = edge_index[0]
  dst = edge_index[1]
  zagg = jnp.zeros((CH, D), jnp.float32)
  agg, deg = _sc_edge_aggregate(x, src, dst, zagg)
  h = _tc_h(x, agg, deg.T, lin_l_w, lin_l_b.reshape(1, HID), lin_r_w)
  pool32 = _sc_pool(h, batch)
  return _tc_head(pool32, batch.reshape(1, N), x,
                  lin0_w, lin0_b.reshape(1, HID),
                  lin1_w, lin1_b.reshape(1, HID),
                  lin2_w, lin2_b.reshape(1, 2))


# speedup vs baseline: 8.9530x; 1.4798x over previous
import dataclasses
import functools

import jax
import jax.numpy as jnp
from jax import lax
from jax.experimental import pallas as pl
from jax.experimental.pallas import tpu as pltpu
from jax.experimental.pallas import tpu_sc as plsc

N = 10000
E = 320000
D = 128
HID = 128
G = 128

NC, NS = 2, 16
NW = NC * NS
EPC = E // NC
EPT = EPC // NS
CH = 80
NCHUNK = EPT // CH
RSTEP = 624
RSZ = 640

PG = 40
PR = PG * 8


def _sc_edge_aggregate(x, src, dst, zagg):
  mesh = plsc.VectorSubcoreMesh(core_axis_name="c", subcore_axis_name="s")
  cp = pltpu.CompilerParams()
  if "needs_layout_passes" in pltpu.CompilerParams.__dataclass_fields__:
    cp = dataclasses.replace(cp, needs_layout_passes=False)

  @functools.partial(
      pl.kernel,
      compiler_params=cp,
      out_type=(jax.ShapeDtypeStruct((NC, N, D), jnp.float32),
                jax.ShapeDtypeStruct((NW, N), jnp.float32)),
      mesh=mesh,
      scratch_types=[
          pltpu.VMEM((CH,), jnp.int32),
          pltpu.VMEM((CH,), jnp.int32),
          pltpu.VMEM((CH,), jnp.int32),
          pltpu.VMEM((CH,), jnp.int32),
          pltpu.VMEM((CH, D), jnp.float32),
          pltpu.VMEM((CH, D), jnp.float32),
          pltpu.VMEM((N,), jnp.float32),
          pltpu.VMEM_SHARED((N, D), jnp.float32),
          pltpu.SemaphoreType.DMA,
          pltpu.SemaphoreType.DMA,
      ],
  )
  def k(x_hbm, src_hbm, dst_hbm, zagg_hbm, agg_hbm, deg_hbm,
        isrc0, isrc1, idst0, idst1, rows0, rows1, degl, sh_agg, sem0, sem1):
    c = lax.axis_index("c")
    s = lax.axis_index("s")
    r0 = s * RSTEP

    @pl.loop(0, N // 16)
    def _(i):
      degl[pl.ds(i * 16, 16)] = jnp.zeros((16,), jnp.float32)

    pltpu.sync_copy(zagg_hbm, rows0)

    @pl.loop(0, RSZ // CH)
    def _(t):
      pltpu.sync_copy(rows0, sh_agg.at[pl.ds(r0 + t * CH, CH)])

    plsc.subcore_barrier()

    base_t = c * EPC + s * EPT
    ones16 = jnp.ones((16,), jnp.float32)

    def load_idx(b, isrc, idst):
      pltpu.sync_copy(dst_hbm.at[pl.ds(b, CH)], idst)
      pltpu.sync_copy(src_hbm.at[pl.ds(b, CH)], isrc)

    def process(isrc, idst, rows, sem):
      pltpu.make_async_copy(x_hbm.at[isrc], rows, sem).wait()
      pltpu.sync_copy(rows, sh_agg.at[idst], add=True)
      for i in range(CH // 16):
        plsc.addupdate_scatter(degl, [idst[pl.ds(i * 16, 16)]], ones16)

    load_idx(base_t, isrc0, idst0)
    pltpu.async_copy(x_hbm.at[isrc0], rows0, sem0)

    @pl.loop(0, (NCHUNK - 1) // 2)
    def _(t):
      b = base_t + 2 * t * CH
      load_idx(b + CH, isrc1, idst1)
      pltpu.async_copy(x_hbm.at[isrc1], rows1, sem1)
      process(isrc0, idst0, rows0, sem0)
      load_idx(b + 2 * CH, isrc0, idst0)
      pltpu.async_copy(x_hbm.at[isrc0], rows0, sem0)
      process(isrc1, idst1, rows1, sem1)

    process(isrc0, idst0, rows0, sem0)

    pltpu.sync_copy(degl, deg_hbm.at[s * NC + c])
    plsc.subcore_barrier()

    @pl.loop(0, RSZ // CH)
    def _(t):
      rr = r0 + t * CH
      pltpu.sync_copy(sh_agg.at[pl.ds(rr, CH)], rows0)
      pltpu.sync_copy(rows0, agg_hbm.at[c, pl.ds(rr, CH)])

  return k(x, src, dst, zagg)


def _tc_h(x, agg, deg, wl, bl, wr):
  BN = 1000

  def body(x_ref, a_ref, d_ref, wl_ref, bl_ref, wr_ref, h_ref):
    a = a_ref[0] + a_ref[1]
    d = jnp.sum(d_ref[...], axis=1)[:, None]
    mean = a / jnp.clip(d, 1.0, None)
    acc = lax.dot_general(mean, wl_ref[...], (((1,), (1,)), ((), ())),
                          preferred_element_type=jnp.float32)
    acc += bl_ref[...]
    acc += lax.dot_general(x_ref[...], wr_ref[...], (((1,), (1,)), ((), ())),
                           preferred_element_type=jnp.float32)
    h_ref[...] = jnp.maximum(acc, 0.0)

  return pl.pallas_call(
      body,
      grid=(N // BN,),
      in_specs=[
          pl.BlockSpec((BN, D), lambda i: (i, 0)),
          pl.BlockSpec((NC, BN, D), lambda i: (0, i, 0)),
          pl.BlockSpec((BN, NW), lambda i: (i, 0)),
          pl.BlockSpec((HID, D), lambda i: (0, 0)),
          pl.BlockSpec((1, HID), lambda i: (0, 0)),
          pl.BlockSpec((HID, D), lambda i: (0, 0)),
      ],
      out_specs=pl.BlockSpec((BN, HID), lambda i: (i, 0)),
      out_shape=jax.ShapeDtypeStruct((N, HID), jnp.float32),
  )(x, agg, deg, wl, bl, wr)


def _sc_pool(h, batch1d):
  mesh = plsc.VectorSubcoreMesh(core_axis_name="c", subcore_axis_name="s")

  @functools.partial(
      pl.kernel,
      out_type=jax.ShapeDtypeStruct((NW, G, HID), jnp.float32),
      mesh=mesh,
      scratch_types=[
          pltpu.VMEM((PR, HID), jnp.float32),
          pltpu.VMEM((G, HID), jnp.float32),
          pltpu.VMEM((PR,), jnp.int32),
      ],
  )
  def k(h_hbm, b_hbm, out_hbm, hbuf, pool, bsm):
    c = lax.axis_index("c")
    s = lax.axis_index("s")
    wid = s * NC + c

    @pl.loop(0, G)
    def _(i):
      @pl.loop(0, HID // 16)
      def _(j):
        pool[i, pl.ds(j * 16, 16)] = jnp.full((16,), -jnp.inf, jnp.float32)

    def scan_rows(nblk):
      @pl.loop(0, nblk)
      def _(bi):
        bv = bsm[pl.ds(bi * 16, 16)]
        for ri in range(16):
          g = bv[ri]
          r = bi * 16 + ri

          @pl.loop(0, HID // 16)
          def _(j):
            v = hbuf[r, pl.ds(j * 16, 16)]
            cur = pool[g, pl.ds(j * 16, 16)]
            pool[g, pl.ds(j * 16, 16)] = jnp.maximum(cur, v)

    g0 = 8 * jnp.minimum(wid * (PG // 8), (1248 - PG) // 8)
    pltpu.sync_copy(h_hbm.at[pl.ds(g0 * 8, PR)], hbuf)
    pltpu.sync_copy(b_hbm.at[pl.ds(g0 * 8, PR)], bsm)
    scan_rows(PR // 16)

    @pl.when(wid == NW - 1)
    def _():
      pltpu.sync_copy(h_hbm.at[pl.ds(1248 * 8, 16)], hbuf.at[pl.ds(0, 16)])
      pltpu.sync_copy(b_hbm.at[pl.ds(1248 * 8, 16)], bsm.at[pl.ds(0, 16)])
      scan_rows(1)

    pltpu.sync_copy(pool, out_hbm.at[wid])

  return k(h, batch1d)


def _tc_head(pool32, batch_row, x, w0, b0, w1, b1, w2, b2):
  def body(p_ref, b_ref, x_ref, w0_ref, b0_ref, w1_ref, b1_ref, w2_ref,
           b2_ref, o_ref):
    pooled = jnp.max(p_ref[...], axis=0)
    b = b_ref[...]
    garange = lax.broadcasted_iota(jnp.int32, (G, 1), 0)
    cnt = jnp.sum((b < garange).astype(jnp.int32), axis=1, keepdims=True)
    root = jnp.clip(cnt, 0, N - 1)
    narange = lax.broadcasted_iota(jnp.int32, (G, N), 1)
    sel = (narange == root).astype(jnp.float32)
    news = lax.dot_general(sel, x_ref[...], (((1,), (0,)), ((), ())),
                           preferred_element_type=jnp.float32)
    news = jnp.maximum(
        lax.dot_general(news, w0_ref[...], (((1,), (1,)), ((), ())),
                        preferred_element_type=jnp.float32) + b0_ref[...],
        0.0)
    cat = jnp.concatenate([news, pooled], axis=-1)
    h1 = jnp.maximum(
        lax.dot_general(cat, w1_ref[...], (((1,), (1,)), ((), ())),
                        preferred_element_type=jnp.float32) + b1_ref[...],
        0.0)
    out = lax.dot_general(h1, w2_ref[...], (((1,), (1,)), ((), ())),
                          preferred_element_type=jnp.float32) + b2_ref[...]
    m = jnp.max(out, axis=-1, keepdims=True)
    z = out - m
    o_ref[...] = z - jnp.log(jnp.sum(jnp.exp(z), axis=-1, keepdims=True))

  return pl.pallas_call(
      body,
      out_shape=jax.ShapeDtypeStruct((G, 2), jnp.float32),
  )(pool32, batch_row, x, w0, b0, w1, b1, w2, b2)


def kernel(x, edge_index, batch, lin_l_w, lin_l_b, lin_r_w,
           lin0_w, lin0_b, lin1_w, lin1_b, lin2_w, lin2_b):
  src = edge_index[0]
  dst = edge_index[1]
  zagg = jnp.zeros((CH, D), jnp.float32)
  agg, deg = _sc_edge_aggregate(x, src, dst, zagg)
  h = _tc_h(x, agg, deg.T, lin_l_w, lin_l_b.reshape(1, HID), lin_r_w)
  pool32 = _sc_pool(h, batch)
  return _tc_head(pool32, batch.reshape(1, N), x,
                  lin0_w, lin0_b.reshape(1, HID),
                  lin1_w, lin1_b.reshape(1, HID),
                  lin2_w, lin2_b.reshape(1, 2))
